# Initial kernel scaffold; baseline (speedup 1.0000x reference)
import jax
import jax.numpy as jnp
from jax.experimental import pallas as pl


def kernel(x, edge_index, batch, W1, b1, W2, b2, Wfc, bfc):
    raise NotImplementedError("write your pallas kernel here")


# speedup vs baseline: 7.9745x; 7.9745x over previous
import functools

import jax
import jax.numpy as jnp
from jax import lax
from jax.experimental import pallas as pl
from jax.experimental.pallas import tpu as pltpu
from jax.experimental.pallas import tpu_sc as plsc

N_NODES_ = 10000
N_PAD = 10240
D_ = 256
HALF = 128
N_GRAPHS_ = 64
N_EDGES_ = 160000
NC = 2
NS = 16
CHUNK = 128
CPT = 79
EPT = CPT * CHUNK
E_PAD = NS * EPT
ROWS_PER_TILE = N_PAD // NS
DUMP_ROW = 10000

_mesh = plsc.VectorSubcoreMesh(core_axis_name="c", subcore_axis_name="s")
_sc_params = pltpu.CompilerParams(
    needs_layout_passes=False, use_tc_tiling_on_sc=False)


@functools.partial(
    pl.kernel, mesh=_mesh, compiler_params=_sc_params,
    out_type=jax.ShapeDtypeStruct((NC * NS, N_PAD), jnp.float32),
    scratch_types=[
        pltpu.VMEM((N_PAD,), jnp.float32),
        pltpu.VMEM((CHUNK,), jnp.int32),
    ],
)
def _deg_kernel(dst_hbm, out_hbm, dl, dstv):
    cid = lax.axis_index("c")
    sid = lax.axis_index("s")
    wid = cid * NS + sid
    zeros = jnp.zeros((16,), jnp.float32)
    ones = jnp.ones((16,), jnp.float32)

    @pl.loop(0, N_PAD // 16)
    def _z(i):
        dl[pl.ds(i * 16, 16)] = zeros

    @pl.loop(0, CPT // 2 + 1)
    def _chunks(ci):
        @pl.when(ci * 2 + cid < CPT)
        def _do():
            base = sid * EPT + (ci * 2 + cid) * CHUNK
            pltpu.sync_copy(dst_hbm.at[pl.ds(base, CHUNK)], dstv)

            @pl.loop(0, CHUNK // 16)
            def _scat(j):
                idx = dstv[pl.ds(j * 16, 16)]
                plsc.addupdate_scatter(dl.at[:], [idx], ones)

    pltpu.sync_copy(dl, out_hbm.at[wid])


def _dinv_body(parts_ref, out_ref):
    deg = jnp.sum(parts_ref[...], axis=0) + 1.0
    out_ref[...] = lax.rsqrt(deg)


def _dinv(parts):
    return pl.pallas_call(
        _dinv_body,
        out_shape=jax.ShapeDtypeStruct((N_PAD,), jnp.float32),
    )(parts)


_BLK = 2048
_NBLK = N_PAD // _BLK


def _mm1_body(x_ref, w_ref, dinv_ref, out_ref):
    h = jnp.dot(x_ref[...], w_ref[...], preferred_element_type=jnp.float32)
    g = h * dinv_ref[...][:, None]
    out_ref[0] = g[:, :HALF]
    out_ref[1] = g[:, HALF:]


def _mm1(x_pad, w1, dinv):
    return pl.pallas_call(
        _mm1_body,
        grid=(_NBLK,),
        in_specs=[
            pl.BlockSpec((_BLK, D_), lambda i: (i, 0)),
            pl.BlockSpec((D_, D_), lambda i: (0, 0)),
            pl.BlockSpec((_BLK,), lambda i: (i,)),
        ],
        out_specs=pl.BlockSpec((2, _BLK, HALF), lambda i: (0, i, 0)),
        out_shape=jax.ShapeDtypeStruct((2, N_PAD, HALF), jnp.float32),
        compiler_params=pltpu.CompilerParams(
            dimension_semantics=("parallel",)),
    )(x_pad, w1, dinv)


@functools.partial(
    pl.kernel, mesh=_mesh, compiler_params=_sc_params,
    out_type=jax.ShapeDtypeStruct((NC * N_PAD, HALF), jnp.float32),
    scratch_types=[
        pltpu.VMEM_SHARED((N_PAD, HALF), jnp.float32),
        pltpu.VMEM((CHUNK,), jnp.int32),
        pltpu.VMEM((CHUNK,), jnp.int32),
        pltpu.VMEM((CHUNK,), jnp.int32),
        pltpu.VMEM((CHUNK, HALF), jnp.float32),
        pltpu.SemaphoreType.DMA,
    ],
)
def _agg_kernel(g_hbm, src_hbm, dst_hbm, out_hbm, acc, srcv, dstv, idxv,
                rows, sem):
    cid = lax.axis_index("c")
    sid = lax.axis_index("s")
    goff = cid * N_PAD

    pltpu.sync_copy(
        g_hbm.at[pl.ds(goff + sid * ROWS_PER_TILE, ROWS_PER_TILE)],
        acc.at[pl.ds(sid * ROWS_PER_TILE, ROWS_PER_TILE)])
    plsc.subcore_barrier()

    @pl.loop(0, CPT)
    def _chunk(ci):
        base = sid * EPT + ci * CHUNK
        pltpu.sync_copy(src_hbm.at[pl.ds(base, CHUNK)], srcv)
        pltpu.sync_copy(dst_hbm.at[pl.ds(base, CHUNK)], dstv)

        @pl.loop(0, CHUNK // 16)
        def _off(j):
            idxv[pl.ds(j * 16, 16)] = srcv[pl.ds(j * 16, 16)] + goff

        pltpu.async_copy(g_hbm.at[idxv], rows, sem).wait()
        pltpu.sync_copy(rows, acc.at[dstv], add=True)

    plsc.subcore_barrier()
    pltpu.sync_copy(
        acc.at[pl.ds(sid * ROWS_PER_TILE, ROWS_PER_TILE)],
        out_hbm.at[pl.ds(goff + sid * ROWS_PER_TILE, ROWS_PER_TILE)])


def _mid_body(s_ref, dinv_ref, b1_ref, w2_ref, out_ref):
    dinv = dinv_ref[...][:, None]
    b1 = b1_ref[...]
    ra = jax.nn.relu(s_ref[0] * dinv + b1[:HALF][None, :])
    rb = jax.nn.relu(s_ref[1] * dinv + b1[HALF:][None, :])
    h2 = (jnp.dot(ra, w2_ref[:HALF, :], preferred_element_type=jnp.float32)
          + jnp.dot(rb, w2_ref[HALF:, :], preferred_element_type=jnp.float32))
    g2 = h2 * dinv
    out_ref[0] = g2[:, :HALF]
    out_ref[1] = g2[:, HALF:]


def _mid(s_split, dinv, b1, w2):
    return pl.pallas_call(
        _mid_body,
        grid=(_NBLK,),
        in_specs=[
            pl.BlockSpec((2, _BLK, HALF), lambda i: (0, i, 0)),
            pl.BlockSpec((_BLK,), lambda i: (i,)),
            pl.BlockSpec((D_,), lambda i: (0,)),
            pl.BlockSpec((D_, D_), lambda i: (0, 0)),
        ],
        out_specs=pl.BlockSpec((2, _BLK, HALF), lambda i: (0, i, 0)),
        out_shape=jax.ShapeDtypeStruct((2, N_PAD, HALF), jnp.float32),
        compiler_params=pltpu.CompilerParams(
            dimension_semantics=("parallel",)),
    )(s_split, dinv, b1, w2)


def _pool_body(s_ref, dinv_ref, b2_ref, batch_ref, wfc_ref, bfc_ref,
               out_ref, pooled, counts):
    i = pl.program_id(0)

    @pl.when(i == 0)
    def _init():
        pooled[...] = jnp.zeros((N_GRAPHS_, D_), jnp.float32)
        counts[...] = jnp.zeros((N_GRAPHS_,), jnp.float32)

    dinv = dinv_ref[...][:, None]
    b2 = b2_ref[...]
    sa = s_ref[0] * dinv + b2[:HALF][None, :]
    sb = s_ref[1] * dinv + b2[HALF:][None, :]
    s_out = jnp.concatenate([sa, sb], axis=1)
    gids = lax.broadcasted_iota(jnp.int32, (_BLK, N_GRAPHS_), 1)
    p = (batch_ref[...][:, None] == gids).astype(jnp.float32)
    pooled[...] += lax.dot_general(
        p, s_out, (((0,), (0,)), ((), ())),
        preferred_element_type=jnp.float32)
    counts[...] += jnp.sum(p, axis=0)

    @pl.when(i == _NBLK - 1)
    def _fin():
        mean = pooled[...] / jnp.maximum(counts[...], 1.0)[:, None]
        logits = (jnp.dot(mean, wfc_ref[...],
                          preferred_element_type=jnp.float32)
                  + bfc_ref[...][None, :])
        out_ref[...] = jax.nn.sigmoid(logits[:, 0])


def _pool(s_split, dinv, b2, batch_pad, wfc, bfc):
    return pl.pallas_call(
        _pool_body,
        grid=(_NBLK,),
        in_specs=[
            pl.BlockSpec((2, _BLK, HALF), lambda i: (0, i, 0)),
            pl.BlockSpec((_BLK,), lambda i: (i,)),
            pl.BlockSpec((D_,), lambda i: (0,)),
            pl.BlockSpec((_BLK,), lambda i: (i,)),
            pl.BlockSpec((D_, 1), lambda i: (0, 0)),
            pl.BlockSpec((1,), lambda i: (0,)),
        ],
        out_specs=pl.BlockSpec((N_GRAPHS_,), lambda i: (0,)),
        out_shape=jax.ShapeDtypeStruct((N_GRAPHS_,), jnp.float32),
        scratch_shapes=[
            pltpu.VMEM((N_GRAPHS_, D_), jnp.float32),
            pltpu.VMEM((N_GRAPHS_,), jnp.float32),
        ],
        compiler_params=pltpu.CompilerParams(
            dimension_semantics=("arbitrary",)),
    )(s_split, dinv, b2, batch_pad, wfc, bfc)


def kernel(x, edge_index, batch, W1, b1, W2, b2, Wfc, bfc):
    src = edge_index[0].astype(jnp.int32)
    dst = edge_index[1].astype(jnp.int32)
    npad = E_PAD - N_EDGES_
    src_p = jnp.concatenate([src, jnp.zeros((npad,), jnp.int32)])
    dst_p = jnp.concatenate([dst, jnp.full((npad,), DUMP_ROW, jnp.int32)])
    x_pad = jnp.pad(x, ((0, N_PAD - N_NODES_), (0, 0)))
    batch_p = jnp.concatenate([
        batch.astype(jnp.int32),
        jnp.full((N_PAD - N_NODES_,), N_GRAPHS_, jnp.int32)])

    deg_parts = _deg_kernel(dst_p)
    dinv = _dinv(deg_parts)
    g1 = _mm1(x_pad, W1, dinv)
    s1 = _agg_kernel(g1.reshape(NC * N_PAD, HALF), src_p, dst_p)
    g2 = _mid(s1.reshape(2, N_PAD, HALF), dinv, b1, W2)
    s2 = _agg_kernel(g2.reshape(NC * N_PAD, HALF), src_p, dst_p)
    return _pool(s2.reshape(2, N_PAD, HALF), dinv, b2, batch_p, Wfc, bfc)


# speedup vs baseline: 9.0684x; 1.1372x over previous
import functools

import jax
import jax.numpy as jnp
from jax import lax
from jax.experimental import pallas as pl
from jax.experimental.pallas import tpu as pltpu
from jax.experimental.pallas import tpu_sc as plsc

N_NODES_ = 10000
N_PAD = 10240
D_ = 256
HALF = 128
N_GRAPHS_ = 64
N_EDGES_ = 160000
NC = 2
NS = 16
CHUNK = 128
CPT = 80
EPT = CPT * CHUNK
E_PAD = NS * EPT
ROWS_PER_TILE = N_PAD // NS
DUMP_ROW = 10000

_mesh = plsc.VectorSubcoreMesh(core_axis_name="c", subcore_axis_name="s")
_sc_params = pltpu.CompilerParams(
    needs_layout_passes=False, use_tc_tiling_on_sc=False)


EPW = E_PAD // (NC * NS)


@functools.partial(
    pl.kernel, mesh=_mesh, compiler_params=_sc_params,
    out_type=jax.ShapeDtypeStruct((NC * NS, N_PAD), jnp.float32),
    scratch_types=[
        pltpu.VMEM((N_PAD,), jnp.float32),
        pltpu.VMEM((EPW,), jnp.int32),
    ],
)
def _deg_kernel(dst_hbm, out_hbm, dl, dstv):
    cid = lax.axis_index("c")
    sid = lax.axis_index("s")
    wid = cid * NS + sid
    zeros = jnp.zeros((16,), jnp.float32)
    ones = jnp.ones((16,), jnp.float32)

    pltpu.sync_copy(dst_hbm.at[pl.ds(wid * EPW, EPW)], dstv)

    @pl.loop(0, N_PAD // 16)
    def _z(i):
        dl[pl.ds(i * 16, 16)] = zeros

    @pl.loop(0, EPW // 16)
    def _scat(j):
        idx = dstv[pl.ds(j * 16, 16)]
        plsc.addupdate_scatter(dl.at[:], [idx], ones)

    pltpu.sync_copy(dl, out_hbm.at[wid])


def _dinv_body(parts_ref, out_ref):
    deg = jnp.sum(parts_ref[...], axis=0) + 1.0
    out_ref[...] = lax.rsqrt(deg)


def _dinv(parts):
    return pl.pallas_call(
        _dinv_body,
        out_shape=jax.ShapeDtypeStruct((N_PAD,), jnp.float32),
    )(parts)


_BLK = 2048
_NBLK = N_PAD // _BLK


def _mm1_body(x_ref, w_ref, dinv_ref, out_ref):
    h = jnp.dot(x_ref[...], w_ref[...], preferred_element_type=jnp.float32)
    g = h * dinv_ref[...][:, None]
    out_ref[0] = g[:, :HALF]
    out_ref[1] = g[:, HALF:]


def _mm1(x_pad, w1, dinv):
    return pl.pallas_call(
        _mm1_body,
        grid=(_NBLK,),
        in_specs=[
            pl.BlockSpec((_BLK, D_), lambda i: (i, 0)),
            pl.BlockSpec((D_, D_), lambda i: (0, 0)),
            pl.BlockSpec((_BLK,), lambda i: (i,)),
        ],
        out_specs=pl.BlockSpec((2, _BLK, HALF), lambda i: (0, i, 0)),
        out_shape=jax.ShapeDtypeStruct((2, N_PAD, HALF), jnp.float32),
        compiler_params=pltpu.CompilerParams(
            dimension_semantics=("parallel",)),
    )(x_pad, w1, dinv)


@functools.partial(
    pl.kernel, mesh=_mesh, compiler_params=_sc_params,
    out_type=jax.ShapeDtypeStruct((NC * N_PAD, HALF), jnp.float32),
    scratch_types=[
        pltpu.VMEM_SHARED((N_PAD, HALF), jnp.float32),
        pltpu.VMEM((CPT // 2, CHUNK), jnp.int32),
        pltpu.VMEM((CPT // 2, CHUNK), jnp.int32),
        pltpu.VMEM((CHUNK, HALF), jnp.float32),
        pltpu.VMEM((CHUNK, HALF), jnp.float32),
        pltpu.SemaphoreType.DMA,
        pltpu.SemaphoreType.DMA,
        pltpu.SemaphoreType.DMA,
        pltpu.SemaphoreType.DMA,
    ],
)
def _agg_kernel(g_hbm, idx_hbm, dst_hbm, out_hbm, acc, idxv, dstv,
                rows_a, rows_b, gs_a, gs_b, ss_a, ss_b):
    cid = lax.axis_index("c")
    sid = lax.axis_index("s")
    goff = cid * N_PAD
    bufs = (rows_a, rows_b)
    gs = (gs_a, gs_b)
    ss = (ss_a, ss_b)
    hcpt = CPT // 2

    pltpu.sync_copy(
        g_hbm.at[pl.ds(goff + sid * ROWS_PER_TILE, ROWS_PER_TILE)],
        acc.at[pl.ds(sid * ROWS_PER_TILE, ROWS_PER_TILE)])
    plsc.subcore_barrier()

    def drain(buf, sem):
        pltpu.make_async_copy(g_hbm.at[pl.ds(0, CHUNK)], buf, sem).wait()

    for h in (0, 1):
        pltpu.sync_copy(
            idx_hbm.at[pl.ds((cid * NS + sid) * CPT + h * hcpt, hcpt)], idxv)
        pltpu.sync_copy(
            dst_hbm.at[pl.ds(sid * CPT + h * hcpt, hcpt)], dstv)
        pltpu.async_copy(g_hbm.at[idxv.at[0]], rows_a, gs_a)

        @pl.loop(0, hcpt // 2)
        def _pair(i):
            for b in (0, 1):
                k = i * 2 + b
                x, y = bufs[b], bufs[1 - b]
                drain(x, gs[b])
                if b == 0:
                    @pl.when(i > 0)
                    def _w():
                        drain(y, ss[1 - b])
                    pltpu.async_copy(g_hbm.at[idxv.at[k + 1]], y, gs[1 - b])
                else:
                    drain(y, ss[1 - b])

                    @pl.when(i < hcpt // 2 - 1)
                    def _g():
                        pltpu.async_copy(
                            g_hbm.at[idxv.at[k + 1]], y, gs[1 - b])
                pltpu.async_copy(x, acc.at[dstv.at[k]], ss[b], add=True)

        drain(rows_b, ss[1])

    plsc.subcore_barrier()
    pltpu.sync_copy(
        acc.at[pl.ds(sid * ROWS_PER_TILE, ROWS_PER_TILE)],
        out_hbm.at[pl.ds(goff + sid * ROWS_PER_TILE, ROWS_PER_TILE)])


def _mid_body(s_ref, dinv_ref, b1_ref, w2_ref, out_ref):
    dinv = dinv_ref[...][:, None]
    b1 = b1_ref[...]
    ra = jax.nn.relu(s_ref[0] * dinv + b1[:HALF][None, :])
    rb = jax.nn.relu(s_ref[1] * dinv + b1[HALF:][None, :])
    h2 = (jnp.dot(ra, w2_ref[:HALF, :], preferred_element_type=jnp.float32)
          + jnp.dot(rb, w2_ref[HALF:, :], preferred_element_type=jnp.float32))
    g2 = h2 * dinv
    out_ref[0] = g2[:, :HALF]
    out_ref[1] = g2[:, HALF:]


def _mid(s_split, dinv, b1, w2):
    return pl.pallas_call(
        _mid_body,
        grid=(_NBLK,),
        in_specs=[
            pl.BlockSpec((2, _BLK, HALF), lambda i: (0, i, 0)),
            pl.BlockSpec((_BLK,), lambda i: (i,)),
            pl.BlockSpec((D_,), lambda i: (0,)),
            pl.BlockSpec((D_, D_), lambda i: (0, 0)),
        ],
        out_specs=pl.BlockSpec((2, _BLK, HALF), lambda i: (0, i, 0)),
        out_shape=jax.ShapeDtypeStruct((2, N_PAD, HALF), jnp.float32),
        compiler_params=pltpu.CompilerParams(
            dimension_semantics=("parallel",)),
    )(s_split, dinv, b1, w2)


def _pool_body(s_ref, dinv_ref, b2_ref, batch_ref, wfc_ref, bfc_ref,
               out_ref, pooled, counts):
    i = pl.program_id(0)

    @pl.when(i == 0)
    def _init():
        pooled[...] = jnp.zeros((N_GRAPHS_, D_), jnp.float32)
        counts[...] = jnp.zeros((N_GRAPHS_,), jnp.float32)

    dinv = dinv_ref[...][:, None]
    b2 = b2_ref[...]
    sa = s_ref[0] * dinv + b2[:HALF][None, :]
    sb = s_ref[1] * dinv + b2[HALF:][None, :]
    s_out = jnp.concatenate([sa, sb], axis=1)
    gids = lax.broadcasted_iota(jnp.int32, (_BLK, N_GRAPHS_), 1)
    p = (batch_ref[...][:, None] == gids).astype(jnp.float32)
    pooled[...] += lax.dot_general(
        p, s_out, (((0,), (0,)), ((), ())),
        preferred_element_type=jnp.float32)
    counts[...] += jnp.sum(p, axis=0)

    @pl.when(i == _NBLK - 1)
    def _fin():
        mean = pooled[...] / jnp.maximum(counts[...], 1.0)[:, None]
        logits = (jnp.dot(mean, wfc_ref[...],
                          preferred_element_type=jnp.float32)
                  + bfc_ref[...][None, :])
        out_ref[...] = jax.nn.sigmoid(logits[:, 0])


def _pool(s_split, dinv, b2, batch_pad, wfc, bfc):
    return pl.pallas_call(
        _pool_body,
        grid=(_NBLK,),
        in_specs=[
            pl.BlockSpec((2, _BLK, HALF), lambda i: (0, i, 0)),
            pl.BlockSpec((_BLK,), lambda i: (i,)),
            pl.BlockSpec((D_,), lambda i: (0,)),
            pl.BlockSpec((_BLK,), lambda i: (i,)),
            pl.BlockSpec((D_, 1), lambda i: (0, 0)),
            pl.BlockSpec((1,), lambda i: (0,)),
        ],
        out_specs=pl.BlockSpec((N_GRAPHS_,), lambda i: (0,)),
        out_shape=jax.ShapeDtypeStruct((N_GRAPHS_,), jnp.float32),
        scratch_shapes=[
            pltpu.VMEM((N_GRAPHS_, D_), jnp.float32),
            pltpu.VMEM((N_GRAPHS_,), jnp.float32),
        ],
        compiler_params=pltpu.CompilerParams(
            dimension_semantics=("arbitrary",)),
    )(s_split, dinv, b2, batch_pad, wfc, bfc)


def kernel(x, edge_index, batch, W1, b1, W2, b2, Wfc, bfc):
    src = edge_index[0].astype(jnp.int32)
    dst = edge_index[1].astype(jnp.int32)
    npad = E_PAD - N_EDGES_
    src_p = jnp.concatenate([src, jnp.zeros((npad,), jnp.int32)])
    dst_p = jnp.concatenate([dst, jnp.full((npad,), DUMP_ROW, jnp.int32)])
    idx2 = jnp.concatenate([src_p, src_p + N_PAD]).reshape(
        2 * NS * CPT, CHUNK)
    dst2 = dst_p.reshape(NS * CPT, CHUNK)
    x_pad = jnp.pad(x, ((0, N_PAD - N_NODES_), (0, 0)))
    batch_p = jnp.concatenate([
        batch.astype(jnp.int32),
        jnp.full((N_PAD - N_NODES_,), N_GRAPHS_, jnp.int32)])

    deg_parts = _deg_kernel(dst_p)
    dinv = _dinv(deg_parts)
    g1 = _mm1(x_pad, W1, dinv)
    s1 = _agg_kernel(g1.reshape(NC * N_PAD, HALF), idx2, dst2)
    g2 = _mid(s1.reshape(2, N_PAD, HALF), dinv, b1, W2)
    s2 = _agg_kernel(g2.reshape(NC * N_PAD, HALF), idx2, dst2)
    return _pool(s2.reshape(2, N_PAD, HALF), dinv, b2, batch_p, Wfc, bfc)


# speedup vs baseline: 9.6924x; 1.0688x over previous
import functools

import jax
import jax.numpy as jnp
from jax import lax
from jax.experimental import pallas as pl
from jax.experimental.pallas import tpu as pltpu
from jax.experimental.pallas import tpu_sc as plsc

N_NODES_ = 10000
N_PAD = 10240
D_ = 256
HALF = 128
N_GRAPHS_ = 64
N_EDGES_ = 160000
NC = 2
NS = 16
CHUNK = 128
CPT = 80
EPT = CPT * CHUNK
E_PAD = NS * EPT
ROWS_PER_TILE = N_PAD // NS
DUMP_ROW = 10000

_mesh = plsc.VectorSubcoreMesh(core_axis_name="c", subcore_axis_name="s")
_sc_params = pltpu.CompilerParams(
    needs_layout_passes=False, use_tc_tiling_on_sc=False)


EPW = E_PAD // (NC * NS)


@functools.partial(
    pl.kernel, mesh=_mesh, compiler_params=_sc_params,
    out_type=jax.ShapeDtypeStruct((NC * NS, N_PAD), jnp.float32),
    scratch_types=[
        pltpu.VMEM((N_PAD,), jnp.float32),
        pltpu.VMEM((EPW,), jnp.int32),
    ],
)
def _deg_kernel(dst_hbm, out_hbm, dl, dstv):
    cid = lax.axis_index("c")
    sid = lax.axis_index("s")
    wid = cid * NS + sid
    zeros = jnp.zeros((16,), jnp.float32)
    ones = jnp.ones((16,), jnp.float32)

    pltpu.sync_copy(dst_hbm.at[pl.ds(wid * EPW, EPW)], dstv)

    @pl.loop(0, N_PAD // 16)
    def _z(i):
        dl[pl.ds(i * 16, 16)] = zeros

    @pl.loop(0, EPW // 16)
    def _scat(j):
        idx = dstv[pl.ds(j * 16, 16)]
        plsc.addupdate_scatter(dl.at[:], [idx], ones)

    pltpu.sync_copy(dl, out_hbm.at[wid])


def _dinv_body(parts_ref, out_ref):
    deg = jnp.sum(parts_ref[...], axis=0) + 1.0
    out_ref[...] = lax.rsqrt(deg)


def _dinv(parts):
    return pl.pallas_call(
        _dinv_body,
        out_shape=jax.ShapeDtypeStruct((N_PAD,), jnp.float32),
    )(parts)


_BLK = 2048
_NBLK = N_PAD // _BLK


def _mm1_body(x_ref, w_ref, dinv_ref, out_ref):
    h = jnp.dot(x_ref[...], w_ref[...], preferred_element_type=jnp.float32)
    g = h * dinv_ref[...][:, None]
    out_ref[0] = g[:, :HALF]
    out_ref[1] = g[:, HALF:]


def _mm1(x_pad, w1, dinv):
    return pl.pallas_call(
        _mm1_body,
        grid=(_NBLK,),
        in_specs=[
            pl.BlockSpec((_BLK, D_), lambda i: (i, 0)),
            pl.BlockSpec((D_, D_), lambda i: (0, 0)),
            pl.BlockSpec((_BLK,), lambda i: (i,)),
        ],
        out_specs=pl.BlockSpec((2, _BLK, HALF), lambda i: (0, i, 0)),
        out_shape=jax.ShapeDtypeStruct((2, N_PAD, HALF), jnp.float32),
        compiler_params=pltpu.CompilerParams(
            dimension_semantics=("parallel",)),
    )(x_pad, w1, dinv)


@functools.partial(
    pl.kernel, mesh=_mesh, compiler_params=_sc_params,
    out_type=jax.ShapeDtypeStruct((NC * N_PAD, HALF), jnp.float32),
    scratch_types=[
        pltpu.VMEM_SHARED((N_PAD, HALF), jnp.float32),
        pltpu.VMEM((CPT // 2, CHUNK), jnp.int32),
        pltpu.VMEM((CPT // 2, CHUNK), jnp.int32),
        pltpu.VMEM((CHUNK, HALF), jnp.float32),
        pltpu.VMEM((CHUNK, HALF), jnp.float32),
        pltpu.SemaphoreType.DMA,
        pltpu.SemaphoreType.DMA,
        pltpu.SemaphoreType.DMA,
        pltpu.SemaphoreType.DMA,
    ],
)
def _agg_kernel(g_hbm, idx_hbm, dst_hbm, out_hbm, acc, idxv, dstv,
                rows_a, rows_b, gs_a, gs_b, ss_a, ss_b):
    cid = lax.axis_index("c")
    sid = lax.axis_index("s")
    goff = cid * N_PAD
    bufs = (rows_a, rows_b)
    gs = (gs_a, gs_b)
    ss = (ss_a, ss_b)
    hcpt = CPT // 2

    pltpu.sync_copy(
        g_hbm.at[pl.ds(goff + sid * ROWS_PER_TILE, ROWS_PER_TILE)],
        acc.at[pl.ds(sid * ROWS_PER_TILE, ROWS_PER_TILE)])
    plsc.subcore_barrier()

    def drain(buf, sem):
        pltpu.make_async_copy(g_hbm.at[pl.ds(0, CHUNK)], buf, sem).wait()

    for h in (0, 1):
        pltpu.sync_copy(
            idx_hbm.at[pl.ds((cid * NS + sid) * CPT + h * hcpt, hcpt)], idxv)
        pltpu.sync_copy(
            dst_hbm.at[pl.ds(sid * CPT + h * hcpt, hcpt)], dstv)
        pltpu.async_copy(g_hbm.at[idxv.at[0]], rows_a, gs_a)

        @pl.loop(0, hcpt // 2)
        def _pair(i):
            for b in (0, 1):
                k = i * 2 + b
                x, y = bufs[b], bufs[1 - b]
                if b == 0:
                    @pl.when(i > 0)
                    def _w():
                        drain(y, ss[1 - b])
                    pltpu.async_copy(g_hbm.at[idxv.at[k + 1]], y, gs[1 - b])
                else:
                    drain(y, ss[1 - b])

                    @pl.when(i < hcpt // 2 - 1)
                    def _g():
                        pltpu.async_copy(
                            g_hbm.at[idxv.at[k + 1]], y, gs[1 - b])
                drain(x, gs[b])
                pltpu.async_copy(x, acc.at[dstv.at[k]], ss[b], add=True)

        drain(rows_b, ss[1])

    plsc.subcore_barrier()
    pltpu.sync_copy(
        acc.at[pl.ds(sid * ROWS_PER_TILE, ROWS_PER_TILE)],
        out_hbm.at[pl.ds(goff + sid * ROWS_PER_TILE, ROWS_PER_TILE)])


def _mid_body(s_ref, dinv_ref, b1_ref, w2_ref, out_ref):
    dinv = dinv_ref[...][:, None]
    b1 = b1_ref[...]
    ra = jax.nn.relu(s_ref[0] * dinv + b1[:HALF][None, :])
    rb = jax.nn.relu(s_ref[1] * dinv + b1[HALF:][None, :])
    h2 = (jnp.dot(ra, w2_ref[:HALF, :], preferred_element_type=jnp.float32)
          + jnp.dot(rb, w2_ref[HALF:, :], preferred_element_type=jnp.float32))
    g2 = h2 * dinv
    out_ref[0] = g2[:, :HALF]
    out_ref[1] = g2[:, HALF:]


def _mid(s_split, dinv, b1, w2):
    return pl.pallas_call(
        _mid_body,
        grid=(_NBLK,),
        in_specs=[
            pl.BlockSpec((2, _BLK, HALF), lambda i: (0, i, 0)),
            pl.BlockSpec((_BLK,), lambda i: (i,)),
            pl.BlockSpec((D_,), lambda i: (0,)),
            pl.BlockSpec((D_, D_), lambda i: (0, 0)),
        ],
        out_specs=pl.BlockSpec((2, _BLK, HALF), lambda i: (0, i, 0)),
        out_shape=jax.ShapeDtypeStruct((2, N_PAD, HALF), jnp.float32),
        compiler_params=pltpu.CompilerParams(
            dimension_semantics=("parallel",)),
    )(s_split, dinv, b1, w2)


def _pool_body(s_ref, dinv_ref, b2_ref, batch_ref, wfc_ref, bfc_ref,
               out_ref, pooled, counts):
    i = pl.program_id(0)

    @pl.when(i == 0)
    def _init():
        pooled[...] = jnp.zeros((N_GRAPHS_, D_), jnp.float32)
        counts[...] = jnp.zeros((N_GRAPHS_,), jnp.float32)

    dinv = dinv_ref[...][:, None]
    b2 = b2_ref[...]
    sa = s_ref[0] * dinv + b2[:HALF][None, :]
    sb = s_ref[1] * dinv + b2[HALF:][None, :]
    s_out = jnp.concatenate([sa, sb], axis=1)
    gids = lax.broadcasted_iota(jnp.int32, (_BLK, N_GRAPHS_), 1)
    p = (batch_ref[...][:, None] == gids).astype(jnp.float32)
    pooled[...] += lax.dot_general(
        p, s_out, (((0,), (0,)), ((), ())),
        preferred_element_type=jnp.float32)
    counts[...] += jnp.sum(p, axis=0)

    @pl.when(i == _NBLK - 1)
    def _fin():
        mean = pooled[...] / jnp.maximum(counts[...], 1.0)[:, None]
        logits = (jnp.dot(mean, wfc_ref[...],
                          preferred_element_type=jnp.float32)
                  + bfc_ref[...][None, :])
        out_ref[...] = jax.nn.sigmoid(logits[:, 0])


def _pool(s_split, dinv, b2, batch_pad, wfc, bfc):
    return pl.pallas_call(
        _pool_body,
        grid=(_NBLK,),
        in_specs=[
            pl.BlockSpec((2, _BLK, HALF), lambda i: (0, i, 0)),
            pl.BlockSpec((_BLK,), lambda i: (i,)),
            pl.BlockSpec((D_,), lambda i: (0,)),
            pl.BlockSpec((_BLK,), lambda i: (i,)),
            pl.BlockSpec((D_, 1), lambda i: (0, 0)),
            pl.BlockSpec((1,), lambda i: (0,)),
        ],
        out_specs=pl.BlockSpec((N_GRAPHS_,), lambda i: (0,)),
        out_shape=jax.ShapeDtypeStruct((N_GRAPHS_,), jnp.float32),
        scratch_shapes=[
            pltpu.VMEM((N_GRAPHS_, D_), jnp.float32),
            pltpu.VMEM((N_GRAPHS_,), jnp.float32),
        ],
        compiler_params=pltpu.CompilerParams(
            dimension_semantics=("arbitrary",)),
    )(s_split, dinv, b2, batch_pad, wfc, bfc)


def kernel(x, edge_index, batch, W1, b1, W2, b2, Wfc, bfc):
    src = edge_index[0].astype(jnp.int32)
    dst = edge_index[1].astype(jnp.int32)
    npad = E_PAD - N_EDGES_
    src_p = jnp.concatenate([src, jnp.zeros((npad,), jnp.int32)])
    dst_p = jnp.concatenate([dst, jnp.full((npad,), DUMP_ROW, jnp.int32)])
    idx2 = jnp.concatenate([src_p, src_p + N_PAD]).reshape(
        2 * NS * CPT, CHUNK)
    dst2 = dst_p.reshape(NS * CPT, CHUNK)
    x_pad = jnp.pad(x, ((0, N_PAD - N_NODES_), (0, 0)))
    batch_p = jnp.concatenate([
        batch.astype(jnp.int32),
        jnp.full((N_PAD - N_NODES_,), N_GRAPHS_, jnp.int32)])

    deg_parts = _deg_kernel(dst_p)
    dinv = _dinv(deg_parts)
    g1 = _mm1(x_pad, W1, dinv)
    s1 = _agg_kernel(g1.reshape(NC * N_PAD, HALF), idx2, dst2)
    g2 = _mid(s1.reshape(2, N_PAD, HALF), dinv, b1, W2)
    s2 = _agg_kernel(g2.reshape(NC * N_PAD, HALF), idx2, dst2)
    return _pool(s2.reshape(2, N_PAD, HALF), dinv, b2, batch_p, Wfc, bfc)


# speedup vs baseline: 9.8369x; 1.0149x over previous
import functools

import jax
import jax.numpy as jnp
from jax import lax
from jax.experimental import pallas as pl
from jax.experimental.pallas import tpu as pltpu
from jax.experimental.pallas import tpu_sc as plsc

N_NODES_ = 10000
N_PAD = 10240
D_ = 256
HALF = 128
N_GRAPHS_ = 64
N_EDGES_ = 160000
NC = 2
NS = 16
CHUNK = 64
CPT = 160
EPT = CPT * CHUNK
E_PAD = NS * EPT
ROWS_PER_TILE = N_PAD // NS
DUMP_ROW = 10000

_mesh = plsc.VectorSubcoreMesh(core_axis_name="c", subcore_axis_name="s")
_sc_params = pltpu.CompilerParams(
    needs_layout_passes=False, use_tc_tiling_on_sc=False)


EPW = E_PAD // (NC * NS)


@functools.partial(
    pl.kernel, mesh=_mesh, compiler_params=_sc_params,
    out_type=jax.ShapeDtypeStruct((NC * NS, N_PAD), jnp.float32),
    scratch_types=[
        pltpu.VMEM((N_PAD,), jnp.float32),
        pltpu.VMEM((EPW,), jnp.int32),
    ],
)
def _deg_kernel(dst_hbm, out_hbm, dl, dstv):
    cid = lax.axis_index("c")
    sid = lax.axis_index("s")
    wid = cid * NS + sid
    zeros = jnp.zeros((16,), jnp.float32)
    ones = jnp.ones((16,), jnp.float32)

    pltpu.sync_copy(dst_hbm.at[pl.ds(wid * EPW, EPW)], dstv)

    @pl.loop(0, N_PAD // 16)
    def _z(i):
        dl[pl.ds(i * 16, 16)] = zeros

    @pl.loop(0, EPW // 16)
    def _scat(j):
        idx = dstv[pl.ds(j * 16, 16)]
        plsc.addupdate_scatter(dl.at[:], [idx], ones)

    pltpu.sync_copy(dl, out_hbm.at[wid])


def _dinv_body(parts_ref, out_ref):
    deg = jnp.sum(parts_ref[...], axis=0) + 1.0
    out_ref[...] = lax.rsqrt(deg)


def _dinv(parts):
    return pl.pallas_call(
        _dinv_body,
        out_shape=jax.ShapeDtypeStruct((N_PAD,), jnp.float32),
    )(parts)


_BLK = 2048
_NBLK = N_PAD // _BLK


def _mm1_body(x_ref, w_ref, dinv_ref, out_ref):
    h = jnp.dot(x_ref[...], w_ref[...], preferred_element_type=jnp.float32)
    g = h * dinv_ref[...][:, None]
    out_ref[0] = g[:, :HALF]
    out_ref[1] = g[:, HALF:]


def _mm1(x_pad, w1, dinv):
    return pl.pallas_call(
        _mm1_body,
        grid=(_NBLK,),
        in_specs=[
            pl.BlockSpec((_BLK, D_), lambda i: (i, 0)),
            pl.BlockSpec((D_, D_), lambda i: (0, 0)),
            pl.BlockSpec((_BLK,), lambda i: (i,)),
        ],
        out_specs=pl.BlockSpec((2, _BLK, HALF), lambda i: (0, i, 0)),
        out_shape=jax.ShapeDtypeStruct((2, N_PAD, HALF), jnp.float32),
        compiler_params=pltpu.CompilerParams(
            dimension_semantics=("parallel",)),
    )(x_pad, w1, dinv)


@functools.partial(
    pl.kernel, mesh=_mesh, compiler_params=_sc_params,
    out_type=jax.ShapeDtypeStruct((NC * N_PAD, HALF), jnp.float32),
    scratch_types=[
        pltpu.VMEM_SHARED((N_PAD, HALF), jnp.float32),
        pltpu.VMEM((CPT // 2, CHUNK), jnp.int32),
        pltpu.VMEM((CPT // 2, CHUNK), jnp.int32),
        pltpu.VMEM((CHUNK, HALF), jnp.float32),
        pltpu.VMEM((CHUNK, HALF), jnp.float32),
        pltpu.VMEM((CHUNK, HALF), jnp.float32),
        pltpu.VMEM((CHUNK, HALF), jnp.float32),
        pltpu.SemaphoreType.DMA,
        pltpu.SemaphoreType.DMA,
        pltpu.SemaphoreType.DMA,
        pltpu.SemaphoreType.DMA,
        pltpu.SemaphoreType.DMA,
        pltpu.SemaphoreType.DMA,
        pltpu.SemaphoreType.DMA,
        pltpu.SemaphoreType.DMA,
    ],
)
def _agg_kernel(g_hbm, idx_hbm, dst_hbm, out_hbm, acc, idxv, dstv,
                r0, r1, r2, r3, g0, g1, g2, g3, s0, s1, s2, s3):
    cid = lax.axis_index("c")
    sid = lax.axis_index("s")
    goff = cid * N_PAD
    bufs = (r0, r1, r2, r3)
    gs = (g0, g1, g2, g3)
    ss = (s0, s1, s2, s3)
    nbuf = 4
    hcpt = CPT // 2

    pltpu.sync_copy(
        g_hbm.at[pl.ds(goff + sid * ROWS_PER_TILE, ROWS_PER_TILE)],
        acc.at[pl.ds(sid * ROWS_PER_TILE, ROWS_PER_TILE)])
    plsc.subcore_barrier()

    def drain(buf, sem):
        pltpu.make_async_copy(g_hbm.at[pl.ds(0, CHUNK)], buf, sem).wait()

    for h in (0, 1):
        pltpu.sync_copy(
            idx_hbm.at[pl.ds((cid * NS + sid) * CPT + h * hcpt, hcpt)], idxv)
        pltpu.sync_copy(
            dst_hbm.at[pl.ds(sid * CPT + h * hcpt, hcpt)], dstv)
        for p in range(nbuf - 1):
            pltpu.async_copy(g_hbm.at[idxv.at[p]], bufs[p], gs[p])

        @pl.loop(0, hcpt // nbuf)
        def _quad(i):
            for b in range(nbuf):
                k = i * nbuf + b
                x = bufs[b]
                w = bufs[(b + nbuf - 1) % nbuf]

                def _free_and_prefetch():
                    drain(w, ss[(b + nbuf - 1) % nbuf])

                def _prefetch():
                    pltpu.async_copy(
                        g_hbm.at[idxv.at[k + nbuf - 1]], w,
                        gs[(b + nbuf - 1) % nbuf])

                if b == 0:
                    @pl.when(i > 0)
                    def _w0():
                        _free_and_prefetch()
                    _prefetch()
                else:
                    _free_and_prefetch()

                    @pl.when(i < hcpt // nbuf - 1)
                    def _g1():
                        _prefetch()
                drain(x, gs[b])
                pltpu.async_copy(x, acc.at[dstv.at[k]], ss[b], add=True)

        drain(bufs[(hcpt - 1) % nbuf], ss[(hcpt - 1) % nbuf])

    plsc.subcore_barrier()
    pltpu.sync_copy(
        acc.at[pl.ds(sid * ROWS_PER_TILE, ROWS_PER_TILE)],
        out_hbm.at[pl.ds(goff + sid * ROWS_PER_TILE, ROWS_PER_TILE)])


def _mid_body(s_ref, dinv_ref, b1_ref, w2_ref, out_ref):
    dinv = dinv_ref[...][:, None]
    b1 = b1_ref[...]
    ra = jax.nn.relu(s_ref[0] * dinv + b1[:HALF][None, :])
    rb = jax.nn.relu(s_ref[1] * dinv + b1[HALF:][None, :])
    h2 = (jnp.dot(ra, w2_ref[:HALF, :], preferred_element_type=jnp.float32)
          + jnp.dot(rb, w2_ref[HALF:, :], preferred_element_type=jnp.float32))
    g2 = h2 * dinv
    out_ref[0] = g2[:, :HALF]
    out_ref[1] = g2[:, HALF:]


def _mid(s_split, dinv, b1, w2):
    return pl.pallas_call(
        _mid_body,
        grid=(_NBLK,),
        in_specs=[
            pl.BlockSpec((2, _BLK, HALF), lambda i: (0, i, 0)),
            pl.BlockSpec((_BLK,), lambda i: (i,)),
            pl.BlockSpec((D_,), lambda i: (0,)),
            pl.BlockSpec((D_, D_), lambda i: (0, 0)),
        ],
        out_specs=pl.BlockSpec((2, _BLK, HALF), lambda i: (0, i, 0)),
        out_shape=jax.ShapeDtypeStruct((2, N_PAD, HALF), jnp.float32),
        compiler_params=pltpu.CompilerParams(
            dimension_semantics=("parallel",)),
    )(s_split, dinv, b1, w2)


def _pool_body(s_ref, dinv_ref, b2_ref, batch_ref, wfc_ref, bfc_ref,
               out_ref, pooled, counts):
    i = pl.program_id(0)

    @pl.when(i == 0)
    def _init():
        pooled[...] = jnp.zeros((N_GRAPHS_, D_), jnp.float32)
        counts[...] = jnp.zeros((N_GRAPHS_,), jnp.float32)

    dinv = dinv_ref[...][:, None]
    b2 = b2_ref[...]
    sa = s_ref[0] * dinv + b2[:HALF][None, :]
    sb = s_ref[1] * dinv + b2[HALF:][None, :]
    s_out = jnp.concatenate([sa, sb], axis=1)
    gids = lax.broadcasted_iota(jnp.int32, (_BLK, N_GRAPHS_), 1)
    p = (batch_ref[...][:, None] == gids).astype(jnp.float32)
    pooled[...] += lax.dot_general(
        p, s_out, (((0,), (0,)), ((), ())),
        preferred_element_type=jnp.float32)
    counts[...] += jnp.sum(p, axis=0)

    @pl.when(i == _NBLK - 1)
    def _fin():
        mean = pooled[...] / jnp.maximum(counts[...], 1.0)[:, None]
        logits = (jnp.dot(mean, wfc_ref[...],
                          preferred_element_type=jnp.float32)
                  + bfc_ref[...][None, :])
        out_ref[...] = jax.nn.sigmoid(logits[:, 0])


def _pool(s_split, dinv, b2, batch_pad, wfc, bfc):
    return pl.pallas_call(
        _pool_body,
        grid=(_NBLK,),
        in_specs=[
            pl.BlockSpec((2, _BLK, HALF), lambda i: (0, i, 0)),
            pl.BlockSpec((_BLK,), lambda i: (i,)),
            pl.BlockSpec((D_,), lambda i: (0,)),
            pl.BlockSpec((_BLK,), lambda i: (i,)),
            pl.BlockSpec((D_, 1), lambda i: (0, 0)),
            pl.BlockSpec((1,), lambda i: (0,)),
        ],
        out_specs=pl.BlockSpec((N_GRAPHS_,), lambda i: (0,)),
        out_shape=jax.ShapeDtypeStruct((N_GRAPHS_,), jnp.float32),
        scratch_shapes=[
            pltpu.VMEM((N_GRAPHS_, D_), jnp.float32),
            pltpu.VMEM((N_GRAPHS_,), jnp.float32),
        ],
        compiler_params=pltpu.CompilerParams(
            dimension_semantics=("arbitrary",)),
    )(s_split, dinv, b2, batch_pad, wfc, bfc)


def kernel(x, edge_index, batch, W1, b1, W2, b2, Wfc, bfc):
    src = edge_index[0].astype(jnp.int32)
    dst = edge_index[1].astype(jnp.int32)
    npad = E_PAD - N_EDGES_
    src_p = jnp.concatenate([src, jnp.zeros((npad,), jnp.int32)])
    dst_p = jnp.concatenate([dst, jnp.full((npad,), DUMP_ROW, jnp.int32)])
    idx2 = jnp.concatenate([src_p, src_p + N_PAD]).reshape(
        2 * NS * CPT, CHUNK)
    dst2 = dst_p.reshape(NS * CPT, CHUNK)
    x_pad = jnp.pad(x, ((0, N_PAD - N_NODES_), (0, 0)))
    batch_p = jnp.concatenate([
        batch.astype(jnp.int32),
        jnp.full((N_PAD - N_NODES_,), N_GRAPHS_, jnp.int32)])

    deg_parts = _deg_kernel(dst_p)
    dinv = _dinv(deg_parts)
    g1 = _mm1(x_pad, W1, dinv)
    s1 = _agg_kernel(g1.reshape(NC * N_PAD, HALF), idx2, dst2)
    g2 = _mid(s1.reshape(2, N_PAD, HALF), dinv, b1, W2)
    s2 = _agg_kernel(g2.reshape(NC * N_PAD, HALF), idx2, dst2)
    return _pool(s2.reshape(2, N_PAD, HALF), dinv, b2, batch_p, Wfc, bfc)
